# Initial kernel scaffold; baseline (speedup 1.0000x reference)
import jax
import jax.numpy as jnp
from jax.experimental import pallas as pl


def kernel(x, weight, bias):
    raise NotImplementedError("write your pallas kernel here")


# speedup vs baseline: 2.3632x; 2.3632x over previous
import jax
import jax.numpy as jnp
from jax.experimental import pallas as pl
from jax.experimental.pallas import tpu as pltpu


def _affine_kernel(x_ref, w_ref, b_ref, o_ref):
    x = x_ref[...].astype(jnp.bfloat16)
    w = w_ref[...].astype(jnp.bfloat16)
    acc = jax.lax.dot_general(
        x, w,
        dimension_numbers=(((1,), (1,)), ((), ())),
        preferred_element_type=jnp.float32)
    o_ref[...] = (acc + b_ref[...]).astype(o_ref.dtype)


def kernel(x, weight, bias):
    n, d = x.shape
    o = weight.shape[0]
    tm = min(512, n)
    return pl.pallas_call(
        _affine_kernel,
        out_shape=jax.ShapeDtypeStruct((n, o), x.dtype),
        grid_spec=pltpu.PrefetchScalarGridSpec(
            num_scalar_prefetch=0,
            grid=(pl.cdiv(n, tm),),
            in_specs=[
                pl.BlockSpec((tm, d), lambda i: (i, 0)),
                pl.BlockSpec((o, d), lambda i: (0, 0)),
                pl.BlockSpec((1, o), lambda i: (0, 0)),
            ],
            out_specs=pl.BlockSpec((tm, o), lambda i: (i, 0)),
        ),
        compiler_params=pltpu.CompilerParams(
            dimension_semantics=("parallel",),
            vmem_limit_bytes=56 << 20),
    )(x, weight, bias.reshape(1, o))
